# Initial kernel scaffold; baseline (speedup 1.0000x reference)
import jax
import jax.numpy as jnp
from jax.experimental import pallas as pl


def kernel(boxes, scores):
    raise NotImplementedError("write your pallas kernel here")


# speedup vs baseline: 19.2573x; 19.2573x over previous
import functools

import jax
import jax.numpy as jnp
from jax import lax
from jax.experimental import pallas as pl
from jax.experimental.pallas import tpu as pltpu

_IOU_T = 0.5
_BLK = 256


def _nms_body(bt_ref, brow_ref, srow_ref, ob_ref, os_ref,
              s_ref, keep_ref, kcol_ref, *, n_pad, blk):
    i = pl.program_id(0)
    base = pl.multiple_of(i * blk, blk)

    @pl.when(i == 0)
    def _():
        keep_ref[...] = jnp.ones_like(keep_ref)

    x1 = brow_ref[:, 0:1]
    y1 = brow_ref[:, 1:2]
    x2 = brow_ref[:, 2:3]
    y2 = brow_ref[:, 3:4]
    cx1 = bt_ref[0:1, :]
    cy1 = bt_ref[1:2, :]
    cx2 = bt_ref[2:3, :]
    cy2 = bt_ref[3:4, :]
    area_r = (x2 - x1) * (y2 - y1)
    area_c = (cx2 - cx1) * (cy2 - cy1)
    iw = jnp.maximum(jnp.minimum(x2, cx2) - jnp.maximum(x1, cx1), 0.0)
    ih = jnp.maximum(jnp.minimum(y2, cy2) - jnp.maximum(y1, cy1), 0.0)
    inter = iw * ih
    union = jnp.maximum(area_r + area_c - inter, 1e-9)
    s_ref[...] = jnp.where(inter > _IOU_T * union, 1.0, 0.0)

    iota_b = lax.broadcasted_iota(jnp.int32, (1, blk), 1)

    def body(r, _):
        kb = keep_ref[:, pl.ds(base, blk)]
        kr = jnp.sum(jnp.where(iota_b == r, kb, 0.0), axis=1,
                     keepdims=True)
        kcol_ref[pl.ds(base + r, 1), :] = kr
        row = s_ref[pl.ds(r, 1), pl.ds(base, blk)]
        sup = jnp.where(iota_b > r, row * kr, 0.0)
        keep_ref[:, pl.ds(base, blk)] = kb * (1.0 - sup)
        return 0

    lax.fori_loop(0, blk, body, 0)

    kcol = kcol_ref[pl.ds(base, blk), :]
    supt = jnp.max(s_ref[...] * kcol, axis=0, keepdims=True)
    iota_n = lax.broadcasted_iota(jnp.int32, (1, n_pad), 1)
    keep_ref[...] = keep_ref[...] * jnp.where(
        iota_n >= base + blk, 1.0 - supt, 1.0)

    ob_ref[...] = brow_ref[...] * kcol
    os_ref[...] = srow_ref[...] * kcol


def kernel(boxes, scores):
    n = boxes.shape[0]
    blk = _BLK
    nb = -(-n // blk)
    n_pad = nb * blk

    order = jnp.argsort(-scores)
    b = jnp.take(boxes, order, axis=0)
    s = jnp.take(scores, order, axis=0)
    bp = jnp.concatenate(
        [b, jnp.zeros((n_pad - n, 4), jnp.float32)], axis=0)
    sp = jnp.concatenate(
        [s, jnp.zeros((n_pad - n,), jnp.float32)], axis=0)[:, None]
    bt = bp.T

    ob, osc = pl.pallas_call(
        functools.partial(_nms_body, n_pad=n_pad, blk=blk),
        grid=(nb,),
        in_specs=[
            pl.BlockSpec((4, n_pad), lambda i: (0, 0)),
            pl.BlockSpec((blk, 4), lambda i: (i, 0)),
            pl.BlockSpec((blk, 1), lambda i: (i, 0)),
        ],
        out_specs=[
            pl.BlockSpec((blk, 4), lambda i: (i, 0)),
            pl.BlockSpec((blk, 1), lambda i: (i, 0)),
        ],
        out_shape=[
            jax.ShapeDtypeStruct((n_pad, 4), jnp.float32),
            jax.ShapeDtypeStruct((n_pad, 1), jnp.float32),
        ],
        scratch_shapes=[
            pltpu.VMEM((blk, n_pad), jnp.float32),
            pltpu.VMEM((1, n_pad), jnp.float32),
            pltpu.VMEM((n_pad, 1), jnp.float32),
        ],
    )(bt, bp, sp)

    return jnp.concatenate([ob[:n], osc[:n]], axis=1)


# speedup vs baseline: 20.5693x; 1.0681x over previous
import functools

import jax
import jax.numpy as jnp
from jax import lax
from jax.experimental import pallas as pl
from jax.experimental.pallas import tpu as pltpu

_IOU_T = 0.5
_BLK = 256


def _nms_body(bt_ref, brow_ref, srow_ref, ob_ref, os_ref,
              s_ref, keep_ref, kcol_ref, *, n_pad, blk):
    i = pl.program_id(0)
    base = pl.multiple_of(i * blk, blk)

    @pl.when(i == 0)
    def _():
        keep_ref[...] = jnp.ones_like(keep_ref)

    x1 = brow_ref[:, 0:1]
    y1 = brow_ref[:, 1:2]
    x2 = brow_ref[:, 2:3]
    y2 = brow_ref[:, 3:4]
    cx1 = bt_ref[0:1, :]
    cy1 = bt_ref[1:2, :]
    cx2 = bt_ref[2:3, :]
    cy2 = bt_ref[3:4, :]
    area_r = (x2 - x1) * (y2 - y1)
    area_c = (cx2 - cx1) * (cy2 - cy1)
    iw = jnp.maximum(jnp.minimum(x2, cx2) - jnp.maximum(x1, cx1), 0.0)
    ih = jnp.maximum(jnp.minimum(y2, cy2) - jnp.maximum(y1, cy1), 0.0)
    inter = iw * ih
    union = jnp.maximum(area_r + area_c - inter, 1e-9)
    s_ref[...] = jnp.where(inter > _IOU_T * union, 1.0, 0.0)

    iota_b = lax.broadcasted_iota(jnp.int32, (1, blk), 1)

    diag = s_ref[:, pl.ds(base, blk)]
    tri_r = lax.broadcasted_iota(jnp.int32, (blk, blk), 0)
    tri_c = lax.broadcasted_iota(jnp.int32, (blk, blk), 1)
    s_ref[:, pl.ds(base, blk)] = jnp.where(tri_c > tri_r, diag, 0.0)

    def body(r, kb):
        kr = jnp.sum(jnp.where(iota_b == r, kb, 0.0), axis=1,
                     keepdims=True)
        kcol_ref[pl.ds(base + r, 1), :] = kr
        row = s_ref[pl.ds(r, 1), pl.ds(base, blk)]
        return kb * (1.0 - row * kr)

    kb = lax.fori_loop(0, blk, body, keep_ref[:, pl.ds(base, blk)])
    keep_ref[:, pl.ds(base, blk)] = kb

    kcol = kcol_ref[pl.ds(base, blk), :]
    supt = jnp.max(s_ref[...] * kcol, axis=0, keepdims=True)
    iota_n = lax.broadcasted_iota(jnp.int32, (1, n_pad), 1)
    keep_ref[...] = keep_ref[...] * jnp.where(
        iota_n >= base + blk, 1.0 - supt, 1.0)

    ob_ref[...] = brow_ref[...] * kcol
    os_ref[...] = srow_ref[...] * kcol


def kernel(boxes, scores):
    n = boxes.shape[0]
    blk = _BLK
    nb = -(-n // blk)
    n_pad = nb * blk

    order = jnp.argsort(-scores)
    b = jnp.take(boxes, order, axis=0)
    s = jnp.take(scores, order, axis=0)
    bp = jnp.concatenate(
        [b, jnp.zeros((n_pad - n, 4), jnp.float32)], axis=0)
    sp = jnp.concatenate(
        [s, jnp.zeros((n_pad - n,), jnp.float32)], axis=0)[:, None]
    bt = bp.T

    ob, osc = pl.pallas_call(
        functools.partial(_nms_body, n_pad=n_pad, blk=blk),
        grid=(nb,),
        in_specs=[
            pl.BlockSpec((4, n_pad), lambda i: (0, 0)),
            pl.BlockSpec((blk, 4), lambda i: (i, 0)),
            pl.BlockSpec((blk, 1), lambda i: (i, 0)),
        ],
        out_specs=[
            pl.BlockSpec((blk, 4), lambda i: (i, 0)),
            pl.BlockSpec((blk, 1), lambda i: (i, 0)),
        ],
        out_shape=[
            jax.ShapeDtypeStruct((n_pad, 4), jnp.float32),
            jax.ShapeDtypeStruct((n_pad, 1), jnp.float32),
        ],
        scratch_shapes=[
            pltpu.VMEM((blk, n_pad), jnp.float32),
            pltpu.VMEM((1, n_pad), jnp.float32),
            pltpu.VMEM((n_pad, 1), jnp.float32),
        ],
    )(bt, bp, sp)

    return jnp.concatenate([ob[:n], osc[:n]], axis=1)


# speedup vs baseline: 78.4885x; 3.8158x over previous
import functools

import jax
import jax.numpy as jnp
from jax import lax
from jax.experimental import pallas as pl
from jax.experimental.pallas import tpu as pltpu

_IOU_T = 0.5
_BLK = 256


def _nms_body(bt_ref, brow_ref, st_ref, ot_ref, os_ref,
              s_ref, keep_ref, *, n_pad, blk):
    i = pl.program_id(0)
    base = pl.multiple_of(i * blk, blk)

    @pl.when(i == 0)
    def _():
        keep_ref[...] = jnp.ones_like(keep_ref)

    x1 = brow_ref[:, 0:1]
    y1 = brow_ref[:, 1:2]
    x2 = brow_ref[:, 2:3]
    y2 = brow_ref[:, 3:4]
    cx1 = bt_ref[0:1, :]
    cy1 = bt_ref[1:2, :]
    cx2 = bt_ref[2:3, :]
    cy2 = bt_ref[3:4, :]
    area_r = (x2 - x1) * (y2 - y1)
    area_c = (cx2 - cx1) * (cy2 - cy1)
    iw = jnp.maximum(jnp.minimum(x2, cx2) - jnp.maximum(x1, cx1), 0.0)
    ih = jnp.maximum(jnp.minimum(y2, cy2) - jnp.maximum(y1, cy1), 0.0)
    inter = iw * ih
    union = jnp.maximum(area_r + area_c - inter, 1e-9)
    s_ref[...] = jnp.where(inter > _IOU_T * union, 1.0, 0.0)

    diag = s_ref[:, pl.ds(base, blk)]
    tri_r = lax.broadcasted_iota(jnp.int32, (blk, blk), 0)
    tri_c = lax.broadcasted_iota(jnp.int32, (blk, blk), 1)
    s_ref[:, pl.ds(base, blk)] = jnp.where(tri_c > tri_r, diag, 0.0)

    keep0 = keep_ref[:, pl.ds(base, blk)]

    def _cond(carry):
        return carry[1]

    def _sweep(carry):
        kb, _ = carry
        cnt = jnp.dot(kb, s_ref[:, pl.ds(base, blk)],
                      preferred_element_type=jnp.float32)
        kb_new = keep0 * jnp.where(cnt > 0.5, 0.0, 1.0)
        changed = jnp.sum(jnp.abs(kb_new - kb)) > 0.0
        return (kb_new, changed)

    kb, _ = lax.while_loop(_cond, _sweep, (keep0, True))
    keep_ref[:, pl.ds(base, blk)] = kb

    cnt_t = jnp.dot(kb, s_ref[...], preferred_element_type=jnp.float32)
    iota_n = lax.broadcasted_iota(jnp.int32, (1, n_pad), 1)
    keep_ref[...] = keep_ref[...] * jnp.where(
        (iota_n >= base + blk) & (cnt_t > 0.5), 0.0, 1.0)

    ot_ref[...] = bt_ref[:, pl.ds(base, blk)] * kb
    os_ref[...] = st_ref[:, pl.ds(base, blk)] * kb


def kernel(boxes, scores):
    n = boxes.shape[0]
    blk = _BLK
    nb = -(-n // blk)
    n_pad = nb * blk

    order = jnp.argsort(-scores)
    b = jnp.take(boxes, order, axis=0)
    s = jnp.take(scores, order, axis=0)
    bp = jnp.concatenate(
        [b, jnp.zeros((n_pad - n, 4), jnp.float32)], axis=0)
    st = jnp.concatenate(
        [s, jnp.zeros((n_pad - n,), jnp.float32)], axis=0)[None, :]
    bt = bp.T

    ot, ost = pl.pallas_call(
        functools.partial(_nms_body, n_pad=n_pad, blk=blk),
        grid=(nb,),
        in_specs=[
            pl.BlockSpec((4, n_pad), lambda i: (0, 0)),
            pl.BlockSpec((blk, 4), lambda i: (i, 0)),
            pl.BlockSpec((1, n_pad), lambda i: (0, 0)),
        ],
        out_specs=[
            pl.BlockSpec((4, blk), lambda i: (0, i)),
            pl.BlockSpec((1, blk), lambda i: (0, i)),
        ],
        out_shape=[
            jax.ShapeDtypeStruct((4, n_pad), jnp.float32),
            jax.ShapeDtypeStruct((1, n_pad), jnp.float32),
        ],
        scratch_shapes=[
            pltpu.VMEM((blk, n_pad), jnp.float32),
            pltpu.VMEM((1, n_pad), jnp.float32),
        ],
    )(bt, bp, st)

    return jnp.concatenate([ot, ost], axis=0).T[:n]
